# Initial kernel scaffold; baseline (speedup 1.0000x reference)
import functools

import jax
import jax.numpy as jnp
from jax import lax
from jax.experimental import pallas as pl
from jax.experimental.pallas import tpu as pltpu
from jax.experimental.pallas import tpu_sc as plsc

N_NODES = 10000
D = 128

NUM_CORES = 2
NUM_SUBCORES = 16
NUM_WORKERS = NUM_CORES * NUM_SUBCORES

CHUNK = 128
CHUNKS_PER_WORKER = 80
EDGES_PADDED = NUM_WORKERS * CHUNKS_PER_WORKER * CHUNK
AGG_ROWS = 10240
ROWS_PER_TEC = AGG_ROWS // NUM_SUBCORES
DUMMY_NODE = N_NODES
DEG_W = 16


def _sc_aggregate_body(x_hbm, row_hbm, col_hbm, zer_hbm, zer16_hbm, ones_hbm,
                       agg_out, deg_out,
                       row_v, col_v, gbuf_a, gbuf_b, ones_v, agg_sp, deg_sp,
                       sem_a, sem_b):
    cid = lax.axis_index("c")
    sid = lax.axis_index("s")
    wid = sid * NUM_CORES + cid

    pltpu.sync_copy(row_hbm.at[wid], row_v)
    pltpu.sync_copy(col_hbm.at[wid], col_v)
    pltpu.sync_copy(ones_hbm, ones_v)
    r0 = sid * ROWS_PER_TEC
    pltpu.sync_copy(zer_hbm, agg_sp.at[pl.ds(r0, ROWS_PER_TEC)])
    pltpu.sync_copy(zer16_hbm, deg_sp.at[pl.ds(r0, ROWS_PER_TEC)])
    plsc.subcore_barrier()

    pltpu.async_copy(x_hbm.at[row_v.at[0]], gbuf_a, sem_a)

    def body(k, _):
        i0 = 2 * k
        i1 = 2 * k + 1
        pltpu.async_copy(x_hbm.at[row_v.at[i1]], gbuf_b, sem_b)
        pltpu.make_async_copy(x_hbm.at[row_v.at[i0]], gbuf_a, sem_a).wait()
        pltpu.sync_copy(gbuf_a, agg_sp.at[col_v.at[i0]], add=True)
        pltpu.sync_copy(ones_v, deg_sp.at[col_v.at[i0]], add=True)
        nxt = jnp.minimum(i0 + 2, CHUNKS_PER_WORKER - 1)
        pltpu.async_copy(x_hbm.at[row_v.at[nxt]], gbuf_a, sem_a)
        pltpu.make_async_copy(x_hbm.at[row_v.at[i1]], gbuf_b, sem_b).wait()
        pltpu.sync_copy(gbuf_b, agg_sp.at[col_v.at[i1]], add=True)
        pltpu.sync_copy(ones_v, deg_sp.at[col_v.at[i1]], add=True)
        return 0

    lax.fori_loop(0, CHUNKS_PER_WORKER // 2, body, 0)
    pltpu.make_async_copy(x_hbm.at[row_v.at[0]], gbuf_a, sem_a).wait()

    plsc.subcore_barrier()
    pltpu.sync_copy(agg_sp.at[pl.ds(r0, ROWS_PER_TEC)],
                    agg_out.at[cid, pl.ds(r0, ROWS_PER_TEC)])
    pltpu.sync_copy(deg_sp.at[pl.ds(r0, ROWS_PER_TEC)],
                    deg_out.at[cid, pl.ds(r0, ROWS_PER_TEC)])


@jax.jit
def _sc_aggregate(x, row3d, col3d, zer, zer16, ones16):
    mesh = plsc.VectorSubcoreMesh(core_axis_name="c", subcore_axis_name="s")
    return pl.kernel(
        _sc_aggregate_body,
        out_type=(
            jax.ShapeDtypeStruct((NUM_CORES, AGG_ROWS, D), jnp.float32),
            jax.ShapeDtypeStruct((NUM_CORES, AGG_ROWS, DEG_W), jnp.float32),
        ),
        mesh=mesh,
        scratch_types=[
            pltpu.VMEM((CHUNKS_PER_WORKER, CHUNK), jnp.int32),
            pltpu.VMEM((CHUNKS_PER_WORKER, CHUNK), jnp.int32),
            pltpu.VMEM((CHUNK, D), jnp.float32),
            pltpu.VMEM((CHUNK, D), jnp.float32),
            pltpu.VMEM((CHUNK, DEG_W), jnp.float32),
            pltpu.VMEM_SHARED((AGG_ROWS, D), jnp.float32),
            pltpu.VMEM_SHARED((AGG_ROWS, DEG_W), jnp.float32),
            pltpu.SemaphoreType.DMA,
            pltpu.SemaphoreType.DMA,
        ],
    )(x, row3d, col3d, zer, zer16, ones16)


ROW_TILE = 400


def _tc_dense_body(x_ref, agg_ref, deg_ref, ws_ref, wd_ref, wf_ref,
                   bs_ref, bd_ref, bf_ref, out_ref):
    xb = x_ref[...]
    aggb = agg_ref[0] + agg_ref[1]
    degb = deg_ref[0, :, 0:1] + deg_ref[1, :, 0:1]
    dn = (((1,), (1,)), ((), ()))
    t_self = lax.dot_general(xb, wf_ref[...], dn, preferred_element_type=jnp.float32)
    t_dst = lax.dot_general(xb, wd_ref[...], dn, preferred_element_type=jnp.float32)
    t_src = lax.dot_general(aggb, ws_ref[...], dn, preferred_element_type=jnp.float32)
    out_ref[...] = (t_self + t_src - degb * t_dst
                    + bf_ref[...] + degb * (bs_ref[...] - bd_ref[...]))


@jax.jit
def _tc_dense(x, agg, deg, W_src, W_dst, W_self, b_src, b_dst, b_self):
    grid = (N_NODES // ROW_TILE,)
    return pl.pallas_call(
        _tc_dense_body,
        grid=grid,
        in_specs=[
            pl.BlockSpec((ROW_TILE, D), lambda i: (i, 0)),
            pl.BlockSpec((NUM_CORES, ROW_TILE, D), lambda i: (0, i, 0)),
            pl.BlockSpec((NUM_CORES, ROW_TILE, DEG_W), lambda i: (0, i, 0)),
            pl.BlockSpec((D, D), lambda i: (0, 0)),
            pl.BlockSpec((D, D), lambda i: (0, 0)),
            pl.BlockSpec((D, D), lambda i: (0, 0)),
            pl.BlockSpec((1, D), lambda i: (0, 0)),
            pl.BlockSpec((1, D), lambda i: (0, 0)),
            pl.BlockSpec((1, D), lambda i: (0, 0)),
        ],
        out_specs=pl.BlockSpec((ROW_TILE, D), lambda i: (i, 0)),
        out_shape=jax.ShapeDtypeStruct((N_NODES, D), jnp.float32),
    )(x, agg, deg, W_src, W_dst, W_self, b_src, b_dst, b_self)


def kernel(x, edge_index, W_src, b_src, W_dst, b_dst, W_self, b_self):
    row = edge_index[0].astype(jnp.int32)
    col = edge_index[1].astype(jnp.int32)
    n_edges = row.shape[0]
    pad = EDGES_PADDED - n_edges
    row_p = jnp.concatenate([row, jnp.zeros((pad,), jnp.int32)])
    col_p = jnp.concatenate([col, jnp.full((pad,), DUMMY_NODE, jnp.int32)])
    row3d = row_p.reshape(NUM_WORKERS, CHUNKS_PER_WORKER, CHUNK)
    col3d = col_p.reshape(NUM_WORKERS, CHUNKS_PER_WORKER, CHUNK)

    zer = jnp.zeros((ROWS_PER_TEC, D), jnp.float32)
    zer16 = jnp.zeros((ROWS_PER_TEC, DEG_W), jnp.float32)
    ones16 = jnp.ones((CHUNK, DEG_W), jnp.float32)

    agg, deg = _sc_aggregate(x, row3d, col3d, zer, zer16, ones16)
    return _tc_dense(x, agg, deg, W_src, W_dst, W_self,
                     b_src.reshape(1, D), b_dst.reshape(1, D),
                     b_self.reshape(1, D))


# speedup vs baseline: 3.9870x; 3.9870x over previous
import jax
import jax.numpy as jnp
from jax import lax
from jax.experimental import pallas as pl
from jax.experimental.pallas import tpu as pltpu
from jax.experimental.pallas import tpu_sc as plsc

N_NODES = 10000
D = 128

NUM_CORES = 2
NUM_SUBCORES = 16
NUM_WORKERS = NUM_CORES * NUM_SUBCORES

CHUNK = 128
CHUNKS_PER_WORKER = 80
EDGES_PADDED = NUM_WORKERS * CHUNKS_PER_WORKER * CHUNK
AGG_ROWS = 10240
ROWS_PER_TEC = AGG_ROWS // NUM_SUBCORES
DUMMY_NODE = N_NODES
DEG_W = 128


def _sc_agg_body(x_hbm, row_hbm, col_hbm, zer_hbm,
                 agg_out,
                 row_v, col_v, gbuf, agg_sp,
                 sem):
    cid = lax.axis_index("c")
    sid = lax.axis_index("s")
    wid = sid * NUM_CORES + cid

    pltpu.sync_copy(row_hbm.at[wid], row_v)
    pltpu.sync_copy(col_hbm.at[wid], col_v)
    r0 = sid * ROWS_PER_TEC
    pltpu.sync_copy(zer_hbm, agg_sp.at[pl.ds(r0, ROWS_PER_TEC)])
    plsc.subcore_barrier()

    def body(i, _):
        pltpu.async_copy(x_hbm.at[row_v.at[i]], gbuf, sem)
        pltpu.make_async_copy(x_hbm.at[row_v.at[i]], gbuf, sem).wait()
        pltpu.sync_copy(gbuf, agg_sp.at[col_v.at[i]], add=True)
        return 0

    lax.fori_loop(0, CHUNKS_PER_WORKER, body, 0)

    plsc.subcore_barrier()
    pltpu.sync_copy(agg_sp.at[pl.ds(r0, ROWS_PER_TEC)],
                    agg_out.at[cid, pl.ds(r0, ROWS_PER_TEC)])


def _sc_deg_body(col_hbm, zer16_hbm, ones_hbm,
                 deg_out,
                 col_v, ones_v, deg_sp):
    cid = lax.axis_index("c")
    sid = lax.axis_index("s")
    wid = sid * NUM_CORES + cid

    pltpu.sync_copy(col_hbm.at[wid], col_v)
    pltpu.sync_copy(ones_hbm, ones_v)
    r0 = sid * ROWS_PER_TEC
    pltpu.sync_copy(zer16_hbm, deg_sp.at[pl.ds(r0, ROWS_PER_TEC)])
    plsc.subcore_barrier()

    def body(i, _):
        pltpu.sync_copy(ones_v, deg_sp.at[col_v.at[i]], add=True)
        return 0

    lax.fori_loop(0, CHUNKS_PER_WORKER, body, 0)

    plsc.subcore_barrier()
    pltpu.sync_copy(deg_sp.at[pl.ds(r0, ROWS_PER_TEC)],
                    deg_out.at[cid, pl.ds(r0, ROWS_PER_TEC)])


@jax.jit
def _sc_aggregate(x, row3d, col3d, zer, zer16, ones16):
    mesh = plsc.VectorSubcoreMesh(core_axis_name="c", subcore_axis_name="s")
    agg = pl.kernel(
        _sc_agg_body,
        out_type=jax.ShapeDtypeStruct((NUM_CORES, AGG_ROWS, D), jnp.float32),
        mesh=mesh,
        scratch_types=[
            pltpu.VMEM((CHUNKS_PER_WORKER, CHUNK), jnp.int32),
            pltpu.VMEM((CHUNKS_PER_WORKER, CHUNK), jnp.int32),
            pltpu.VMEM((CHUNK, D), jnp.float32),
            pltpu.VMEM_SHARED((AGG_ROWS, D), jnp.float32),
            pltpu.SemaphoreType.DMA,
        ],
    )(x, row3d, col3d, zer)
    deg = pl.kernel(
        _sc_deg_body,
        out_type=jax.ShapeDtypeStruct((NUM_CORES, AGG_ROWS, DEG_W), jnp.float32),
        mesh=mesh,
        scratch_types=[
            pltpu.VMEM((CHUNKS_PER_WORKER, CHUNK), jnp.int32),
            pltpu.VMEM((CHUNK, DEG_W), jnp.float32),
            pltpu.VMEM_SHARED((AGG_ROWS, DEG_W), jnp.float32),
        ],
    )(col3d, zer16, ones16)
    return agg, deg


ROW_TILE = 400


def _tc_dense_body(x_ref, agg_ref, deg_ref, ws_ref, wd_ref, wf_ref,
                   bs_ref, bd_ref, bf_ref, out_ref):
    xb = x_ref[...]
    aggb = agg_ref[0] + agg_ref[1]
    degb = deg_ref[0, :, 0:1] + deg_ref[1, :, 0:1]
    dn = (((1,), (1,)), ((), ()))
    t_self = lax.dot_general(xb, wf_ref[...], dn, preferred_element_type=jnp.float32)
    t_dst = lax.dot_general(xb, wd_ref[...], dn, preferred_element_type=jnp.float32)
    t_src = lax.dot_general(aggb, ws_ref[...], dn, preferred_element_type=jnp.float32)
    out_ref[...] = (t_self + t_src - degb * t_dst
                    + bf_ref[...] + degb * (bs_ref[...] - bd_ref[...]))


@jax.jit
def _tc_dense(x, agg, deg, W_src, W_dst, W_self, b_src, b_dst, b_self):
    grid = (N_NODES // ROW_TILE,)
    return pl.pallas_call(
        _tc_dense_body,
        grid=grid,
        in_specs=[
            pl.BlockSpec((ROW_TILE, D), lambda i: (i, 0)),
            pl.BlockSpec((NUM_CORES, ROW_TILE, D), lambda i: (0, i, 0)),
            pl.BlockSpec((NUM_CORES, ROW_TILE, DEG_W), lambda i: (0, i, 0)),
            pl.BlockSpec((D, D), lambda i: (0, 0)),
            pl.BlockSpec((D, D), lambda i: (0, 0)),
            pl.BlockSpec((D, D), lambda i: (0, 0)),
            pl.BlockSpec((1, D), lambda i: (0, 0)),
            pl.BlockSpec((1, D), lambda i: (0, 0)),
            pl.BlockSpec((1, D), lambda i: (0, 0)),
        ],
        out_specs=pl.BlockSpec((ROW_TILE, D), lambda i: (i, 0)),
        out_shape=jax.ShapeDtypeStruct((N_NODES, D), jnp.float32),
    )(x, agg, deg, W_src, W_dst, W_self, b_src, b_dst, b_self)


def kernel(x, edge_index, W_src, b_src, W_dst, b_dst, W_self, b_self):
    row = edge_index[0].astype(jnp.int32)
    col = edge_index[1].astype(jnp.int32)
    pad = EDGES_PADDED - row.shape[0]
    row_p = jnp.concatenate([row, jnp.zeros((pad,), jnp.int32)])
    col_p = jnp.concatenate([col, jnp.full((pad,), DUMMY_NODE, jnp.int32)])
    row3d = row_p.reshape(NUM_WORKERS, CHUNKS_PER_WORKER, CHUNK)
    col3d = col_p.reshape(NUM_WORKERS, CHUNKS_PER_WORKER, CHUNK)

    zer = jnp.zeros((ROWS_PER_TEC, D), jnp.float32)
    zer16 = jnp.zeros((ROWS_PER_TEC, DEG_W), jnp.float32)
    ones16 = jnp.ones((CHUNK, DEG_W), jnp.float32)

    agg, deg = _sc_aggregate(x, row3d, col3d, zer, zer16, ones16)
    return _tc_dense(x, agg, deg, W_src, W_dst, W_self,
                     b_src.reshape(1, D), b_dst.reshape(1, D),
                     b_self.reshape(1, D))


# speedup vs baseline: 4.2288x; 1.0607x over previous
import jax
import jax.numpy as jnp
from jax import lax
from jax.experimental import pallas as pl
from jax.experimental.pallas import tpu as pltpu
from jax.experimental.pallas import tpu_sc as plsc

N_NODES = 10000
D = 128

NUM_CORES = 2
NUM_SUBCORES = 16
NUM_WORKERS = NUM_CORES * NUM_SUBCORES

CHUNK = 128
TOTAL_CHUNKS = 2560
EDGES_PADDED = TOTAL_CHUNKS * CHUNK
N0 = 118
N1 = 42
MAXCH = max(N0, N1)
AGG_ROWS = 10240
ROWS_PER_TEC = AGG_ROWS // NUM_SUBCORES
DUMMY_NODE = N_NODES
DEG_W = 128


def _sc_agg_body(x_hbm, row_hbm, col_hbm, zer_hbm,
                 agg_out,
                 row_v, col_v, gbuf, agg_sp,
                 sem):
    cid = lax.axis_index("c")
    sid = lax.axis_index("s")
    wid = sid * NUM_CORES + cid

    pltpu.sync_copy(row_hbm.at[wid], row_v)
    pltpu.sync_copy(col_hbm.at[wid], col_v)
    r0 = sid * ROWS_PER_TEC
    pltpu.sync_copy(zer_hbm, agg_sp.at[pl.ds(r0, ROWS_PER_TEC)])
    plsc.subcore_barrier()

    nchunks = jnp.where(cid == 0, N0, N1)

    def body(i, _):
        pltpu.async_copy(x_hbm.at[row_v.at[i]], gbuf, sem)
        pltpu.make_async_copy(x_hbm.at[row_v.at[i]], gbuf, sem).wait()
        pltpu.sync_copy(gbuf, agg_sp.at[col_v.at[i]], add=True)
        return 0

    lax.fori_loop(0, nchunks, body, 0)

    plsc.subcore_barrier()
    pltpu.sync_copy(agg_sp.at[pl.ds(r0, ROWS_PER_TEC)],
                    agg_out.at[cid, pl.ds(r0, ROWS_PER_TEC)])


def _sc_deg_body(col_hbm, zer16_hbm, ones_hbm,
                 deg_out,
                 col_v, ones_v, deg_sp):
    cid = lax.axis_index("c")
    sid = lax.axis_index("s")
    wid = sid * NUM_CORES + cid

    pltpu.sync_copy(col_hbm.at[wid], col_v)
    pltpu.sync_copy(ones_hbm, ones_v)
    r0 = sid * ROWS_PER_TEC
    pltpu.sync_copy(zer16_hbm, deg_sp.at[pl.ds(r0, ROWS_PER_TEC)])
    plsc.subcore_barrier()

    nchunks = jnp.where(cid == 0, N0, N1)

    def body(i, _):
        pltpu.sync_copy(ones_v, deg_sp.at[col_v.at[i]], add=True)
        return 0

    lax.fori_loop(0, nchunks, body, 0)

    plsc.subcore_barrier()
    pltpu.sync_copy(deg_sp.at[pl.ds(r0, ROWS_PER_TEC)],
                    deg_out.at[cid, pl.ds(r0, ROWS_PER_TEC)])


@jax.jit
def _sc_aggregate(x, row3d, col3d, zer, zer16, ones16):
    mesh = plsc.VectorSubcoreMesh(core_axis_name="c", subcore_axis_name="s")
    agg = pl.kernel(
        _sc_agg_body,
        out_type=jax.ShapeDtypeStruct((NUM_CORES, AGG_ROWS, D), jnp.float32),
        mesh=mesh,
        scratch_types=[
            pltpu.VMEM((MAXCH, CHUNK), jnp.int32),
            pltpu.VMEM((MAXCH, CHUNK), jnp.int32),
            pltpu.VMEM((CHUNK, D), jnp.float32),
            pltpu.VMEM_SHARED((AGG_ROWS, D), jnp.float32),
            pltpu.SemaphoreType.DMA,
        ],
    )(x, row3d, col3d, zer)
    deg = pl.kernel(
        _sc_deg_body,
        out_type=jax.ShapeDtypeStruct((NUM_CORES, AGG_ROWS, DEG_W), jnp.float32),
        mesh=mesh,
        scratch_types=[
            pltpu.VMEM((MAXCH, CHUNK), jnp.int32),
            pltpu.VMEM((CHUNK, DEG_W), jnp.float32),
            pltpu.VMEM_SHARED((AGG_ROWS, DEG_W), jnp.float32),
        ],
    )(col3d, zer16, ones16)
    return agg, deg


ROW_TILE = 400


def _tc_dense_body(x_ref, agg_ref, deg_ref, ws_ref, wd_ref, wf_ref,
                   bs_ref, bd_ref, bf_ref, out_ref):
    xb = x_ref[...]
    aggb = agg_ref[0] + agg_ref[1]
    degb = deg_ref[0, :, 0:1] + deg_ref[1, :, 0:1]
    dn = (((1,), (1,)), ((), ()))
    t_self = lax.dot_general(xb, wf_ref[...], dn, preferred_element_type=jnp.float32)
    t_dst = lax.dot_general(xb, wd_ref[...], dn, preferred_element_type=jnp.float32)
    t_src = lax.dot_general(aggb, ws_ref[...], dn, preferred_element_type=jnp.float32)
    out_ref[...] = (t_self + t_src - degb * t_dst
                    + bf_ref[...] + degb * (bs_ref[...] - bd_ref[...]))


@jax.jit
def _tc_dense(x, agg, deg, W_src, W_dst, W_self, b_src, b_dst, b_self):
    grid = (N_NODES // ROW_TILE,)
    return pl.pallas_call(
        _tc_dense_body,
        grid=grid,
        in_specs=[
            pl.BlockSpec((ROW_TILE, D), lambda i: (i, 0)),
            pl.BlockSpec((NUM_CORES, ROW_TILE, D), lambda i: (0, i, 0)),
            pl.BlockSpec((NUM_CORES, ROW_TILE, DEG_W), lambda i: (0, i, 0)),
            pl.BlockSpec((D, D), lambda i: (0, 0)),
            pl.BlockSpec((D, D), lambda i: (0, 0)),
            pl.BlockSpec((D, D), lambda i: (0, 0)),
            pl.BlockSpec((1, D), lambda i: (0, 0)),
            pl.BlockSpec((1, D), lambda i: (0, 0)),
            pl.BlockSpec((1, D), lambda i: (0, 0)),
        ],
        out_specs=pl.BlockSpec((ROW_TILE, D), lambda i: (i, 0)),
        out_shape=jax.ShapeDtypeStruct((N_NODES, D), jnp.float32),
    )(x, agg, deg, W_src, W_dst, W_self, b_src, b_dst, b_self)


def _route_chunks(flat, fill):
    chunks = flat.reshape(TOTAL_CHUNKS, CHUNK)
    c0 = chunks[:NUM_SUBCORES * N0].reshape(NUM_SUBCORES, N0, CHUNK)
    c1 = chunks[NUM_SUBCORES * N0:].reshape(NUM_SUBCORES, N1, CHUNK)
    if N0 < MAXCH:
        c0 = jnp.pad(c0, ((0, 0), (0, MAXCH - N0), (0, 0)), constant_values=fill)
    if N1 < MAXCH:
        c1 = jnp.pad(c1, ((0, 0), (0, MAXCH - N1), (0, 0)), constant_values=fill)
    return jnp.stack([c0, c1], axis=1).reshape(NUM_WORKERS, MAXCH, CHUNK)


def kernel(x, edge_index, W_src, b_src, W_dst, b_dst, W_self, b_self):
    row = edge_index[0].astype(jnp.int32)
    col = edge_index[1].astype(jnp.int32)
    pad = EDGES_PADDED - row.shape[0]
    row_p = jnp.concatenate([row, jnp.zeros((pad,), jnp.int32)])
    col_p = jnp.concatenate([col, jnp.full((pad,), DUMMY_NODE, jnp.int32)])
    row3d = _route_chunks(row_p, 0)
    col3d = _route_chunks(col_p, DUMMY_NODE)

    zer = jnp.zeros((ROWS_PER_TEC, D), jnp.float32)
    zer16 = jnp.zeros((ROWS_PER_TEC, DEG_W), jnp.float32)
    ones16 = jnp.ones((CHUNK, DEG_W), jnp.float32)

    agg, deg = _sc_aggregate(x, row3d, col3d, zer, zer16, ones16)
    return _tc_dense(x, agg, deg, W_src, W_dst, W_self,
                     b_src.reshape(1, D), b_dst.reshape(1, D),
                     b_self.reshape(1, D))
